# Initial kernel scaffold; baseline (speedup 1.0000x reference)
import jax
import jax.numpy as jnp
from jax.experimental import pallas as pl


def kernel(x, edge_index, W1, b1, gamma, beta, a, W2, b2):
    raise NotImplementedError("write your pallas kernel here")


# speedup vs baseline: 12.9426x; 12.9426x over previous
import functools

import jax
import jax.numpy as jnp
from jax import lax
from jax.experimental import pallas as pl
from jax.experimental.pallas import tpu as pltpu
from jax.experimental.pallas import tpu_sc as plsc

N = 10000
E = 320000
IN_DIM = 128
HID2 = 256
OUT_DIM = 128

NC = 2
NS = 16
L = 16

NPAD = 10240
ROWS = NPAD // NS
CH = 128
CHUNKS = 158
EPAD = NS * CHUNKS * CH


def _sc_mesh():
    return plsc.VectorSubcoreMesh(core_axis_name="c", subcore_axis_name="s")


@functools.partial(
    pl.kernel,
    out_type=jax.ShapeDtypeStruct((NC * NPAD,), jnp.float32),
    mesh=_sc_mesh(),
    scratch_types=[
        pltpu.VMEM((CH,), jnp.int32),
        pltpu.VMEM((CH,), jnp.float32),
        pltpu.VMEM((ROWS,), jnp.float32),
        pltpu.VMEM_SHARED((NPAD,), jnp.float32),
    ],
)
def _deg_call(dst_hbm, out_hbm, dst_v, ones_v, zb, deg_sh):
    c = lax.axis_index("c")
    s = lax.axis_index("s")
    base = s * ROWS
    for i in range(CH // L):
        ones_v[pl.ds(i * L, L)] = jnp.ones((L,), jnp.float32)
    for i in range(ROWS // L):
        zb[pl.ds(i * L, L)] = jnp.zeros((L,), jnp.float32)
    pltpu.sync_copy(zb, deg_sh.at[pl.ds(base, ROWS)])
    plsc.subcore_barrier()

    def _step(j, carry):
        pltpu.sync_copy(dst_hbm.at[s * CHUNKS + j], dst_v)
        pltpu.sync_copy(ones_v, deg_sh.at[dst_v], add=True)
        return carry

    lax.fori_loop(0, CHUNKS, _step, 0)
    plsc.subcore_barrier()
    pltpu.sync_copy(deg_sh.at[pl.ds(base, ROWS)], zb)
    pltpu.sync_copy(zb, out_hbm.at[pl.ds(c * NPAD + base, ROWS)])


def _make_msg(F2, edge_split):
    @functools.partial(
        pl.kernel,
        out_type=jax.ShapeDtypeStruct((NC * NPAD, F2), jnp.float32),
        mesh=_sc_mesh(),
        scratch_types=[
            pltpu.VMEM((CH,), jnp.int32),
            pltpu.VMEM((CH,), jnp.int32),
            pltpu.VMEM((CH, F2), jnp.float32),
            pltpu.VMEM_SHARED((NPAD, F2), jnp.float32),
            pltpu.SemaphoreType.DMA,
        ],
    )
    def _msg(h_hbm, src_hbm, dst_hbm, out_hbm, src_v, dst_v, buf, acc_sh, sem):
        c = lax.axis_index("c")
        s = lax.axis_index("s")
        base = s * ROWS
        if edge_split:
            n_chunks = CHUNKS // NC
            srow0 = drow0 = (c * NS + s) * n_chunks
        else:
            n_chunks = CHUNKS
            srow0 = (c * NS + s) * CHUNKS
            drow0 = s * CHUNKS

        def _zrow(i, carry):
            for f in range(F2 // L):
                buf[i, pl.ds(f * L, L)] = jnp.zeros((L,), jnp.float32)
            return carry

        lax.fori_loop(0, CH, _zrow, 0)
        for r in range(ROWS // CH):
            pltpu.sync_copy(buf, acc_sh.at[pl.ds(base + r * CH, CH), :])
        plsc.subcore_barrier()

        def _step(j, carry):
            pltpu.sync_copy(src_hbm.at[srow0 + j], src_v)
            pltpu.sync_copy(dst_hbm.at[drow0 + j], dst_v)
            pltpu.async_copy(h_hbm.at[src_v], buf, sem).wait()
            pltpu.sync_copy(buf, acc_sh.at[dst_v], add=True)
            return carry

        lax.fori_loop(0, n_chunks, _step, 0)
        plsc.subcore_barrier()
        obase = c * NPAD + base
        for r in range(ROWS // CH):
            pltpu.sync_copy(acc_sh.at[pl.ds(base + r * CH, CH), :], buf)
            pltpu.sync_copy(buf, out_hbm.at[pl.ds(obase + r * CH, CH), :])

    return _msg


_msg1 = _make_msg(HID2 // 2, edge_split=False)
_msg2 = _make_msg(OUT_DIM, edge_split=True)


def _tc1_body(x_ref, w1_ref, deg_ref, out_ref):
    h = jnp.dot(x_ref[...], w1_ref[...], preferred_element_type=jnp.float32)
    dinv = lax.rsqrt(deg_ref[...] + 1.0)
    hs = h * dinv
    out_ref[0:N, :] = hs[:, 0:HID2 // 2]
    out_ref[N:2 * N, :] = hs[:, HID2 // 2:HID2]


def _tc2_body(acc_ref, h_ref, deg_ref, g_ref, be_ref, a_ref, w2_ref, b1_ref,
              out_ref):
    F = HID2 // 2
    dinv = lax.rsqrt(deg_ref[...] + 1.0)
    a = a_ref[0, 0]
    w2 = w2_ref[...]
    h2 = None
    for k in range(2):
        z = (acc_ref[k * NPAD:k * NPAD + N, :] + h_ref[k * N:(k + 1) * N, :])
        z = z * dinv + b1_ref[k:k + 1, :]
        mean = jnp.sum(z, axis=0, keepdims=True) * (1.0 / N)
        d = z - mean
        var = jnp.sum(d * d, axis=0, keepdims=True) * (1.0 / N)
        zn = d * lax.rsqrt(var + 1e-5) * g_ref[k:k + 1, :] + be_ref[k:k + 1, :]
        zp = jnp.where(zn > 0, zn, a * zn)
        part = jnp.dot(zp, w2[k], preferred_element_type=jnp.float32)
        h2 = part if h2 is None else h2 + part
    out_ref[...] = h2 * dinv


def _tc3_body(acc_ref, h_ref, deg_ref, b2_ref, out_ref):
    dinv = lax.rsqrt(deg_ref[...] + 1.0)
    total = acc_ref[0:N, :] + acc_ref[NPAD:NPAD + N, :] + h_ref[...]
    out_ref[...] = total * dinv + b2_ref[...]


_tc1 = pl.pallas_call(
    _tc1_body,
    out_shape=jax.ShapeDtypeStruct((2 * N, HID2 // 2), jnp.float32),
)
_tc2 = pl.pallas_call(
    _tc2_body,
    out_shape=jax.ShapeDtypeStruct((N, OUT_DIM), jnp.float32),
)
_tc3 = pl.pallas_call(
    _tc3_body,
    out_shape=jax.ShapeDtypeStruct((N, OUT_DIM), jnp.float32),
)


def kernel(x, edge_index, W1, b1, gamma, beta, a, W2, b2):
    src = edge_index[0].astype(jnp.int32)
    dst = edge_index[1].astype(jnp.int32)
    P = EPAD - E
    pad = jnp.arange(P, dtype=jnp.int32)
    src_p = jnp.concatenate([src, pad % N]).reshape(NS * CHUNKS, CH)
    dst_p = jnp.concatenate([dst, N + pad % (NPAD - N)]).reshape(NS * CHUNKS, CH)
    src_both = jnp.concatenate([src_p, src_p + N]).reshape(NC * NS * CHUNKS, CH)

    deg = _deg_call(dst_p)[:N].reshape(N, 1)
    h1s = _tc1(x, W1, deg)
    acc1 = _msg1(h1s, src_both, dst_p)
    h2s = _tc2(acc1, h1s, deg, gamma.reshape(2, HID2 // 2),
               beta.reshape(2, HID2 // 2), a.reshape(1, 1),
               W2.reshape(2, HID2 // 2, OUT_DIM), b1.reshape(2, HID2 // 2))
    acc2 = _msg2(h2s, src_p, dst_p)
    return _tc3(acc2, h2s, deg, b2.reshape(1, OUT_DIM))


# speedup vs baseline: 26.6027x; 2.0554x over previous
import functools

import jax
import jax.numpy as jnp
from jax import lax
from jax.experimental import pallas as pl
from jax.experimental.pallas import tpu as pltpu
from jax.experimental.pallas import tpu_sc as plsc

N = 10000
E = 320000
IN_DIM = 128
HID2 = 256
OUT_DIM = 128

NC = 2
NS = 16
L = 16

NPAD = 10240
ROWS = NPAD // NS
CH = 128
CHUNKS = 160
IB = 16
EPAD = NS * CHUNKS * CH


def _sc_mesh():
    return plsc.VectorSubcoreMesh(core_axis_name="c", subcore_axis_name="s")


@functools.partial(
    pl.kernel,
    out_type=jax.ShapeDtypeStruct((NC * NPAD,), jnp.float32),
    mesh=_sc_mesh(),
    scratch_types=[
        pltpu.VMEM((IB, CH), jnp.int32),
        pltpu.VMEM((CH,), jnp.float32),
        pltpu.VMEM((ROWS,), jnp.float32),
        pltpu.VMEM_SHARED((NPAD,), jnp.float32),
        pltpu.SemaphoreType.DMA,
        pltpu.SemaphoreType.DMA,
    ],
)
def _deg_call(dst_hbm, out_hbm, dst_v, ones_v, zb, deg_sh, sem0, sem1):
    c = lax.axis_index("c")
    s = lax.axis_index("s")
    base = s * ROWS
    for i in range(CH // L):
        ones_v[pl.ds(i * L, L)] = jnp.ones((L,), jnp.float32)
    for i in range(ROWS // L):
        zb[pl.ds(i * L, L)] = jnp.zeros((L,), jnp.float32)
    pltpu.sync_copy(zb, deg_sh.at[pl.ds(base, ROWS)])
    plsc.subcore_barrier()
    sems = (sem0, sem1)

    def _block(b, carry):
        pltpu.sync_copy(dst_hbm.at[pl.ds(s * CHUNKS + b * IB, IB), :], dst_v)
        descs = [None, None]
        for i in range(IB):
            p = i % 2
            if descs[p] is not None:
                descs[p].wait()
            descs[p] = pltpu.async_copy(ones_v, deg_sh.at[dst_v.at[i]],
                                        sems[p], add=True)
        descs[0].wait()
        descs[1].wait()
        return carry

    lax.fori_loop(0, CHUNKS // IB, _block, 0)
    plsc.subcore_barrier()
    pltpu.sync_copy(deg_sh.at[pl.ds(base, ROWS)], zb)
    pltpu.sync_copy(zb, out_hbm.at[pl.ds(c * NPAD + base, ROWS)])


def _make_msg(F2, edge_split):
    @functools.partial(
        pl.kernel,
        out_type=jax.ShapeDtypeStruct((NC * NPAD, F2), jnp.float32),
        mesh=_sc_mesh(),
        scratch_types=[
            pltpu.VMEM((IB, CH), jnp.int32),
            pltpu.VMEM((IB, CH), jnp.int32),
            pltpu.VMEM((CH, F2), jnp.float32),
            pltpu.VMEM((CH, F2), jnp.float32),
            pltpu.VMEM_SHARED((NPAD, F2), jnp.float32),
            pltpu.SemaphoreType.DMA,
            pltpu.SemaphoreType.DMA,
            pltpu.SemaphoreType.DMA,
            pltpu.SemaphoreType.DMA,
        ],
    )
    def _msg(h_hbm, src_hbm, dst_hbm, out_hbm, src_v, dst_v, buf_a, buf_b,
             acc_sh, sg0, sg1, ss0, ss1):
        c = lax.axis_index("c")
        s = lax.axis_index("s")
        base = s * ROWS
        if edge_split:
            n_chunks = CHUNKS // NC
            srow0 = drow0 = (c * NS + s) * n_chunks
        else:
            n_chunks = CHUNKS
            srow0 = (c * NS + s) * CHUNKS
            drow0 = s * CHUNKS
        bufs = (buf_a, buf_b)
        sgs = (sg0, sg1)
        sss = (ss0, ss1)

        def _zrow(i, carry):
            for f in range(F2 // L):
                buf_a[i, pl.ds(f * L, L)] = jnp.zeros((L,), jnp.float32)
            return carry

        lax.fori_loop(0, CH, _zrow, 0)
        for r in range(ROWS // CH):
            pltpu.sync_copy(buf_a, acc_sh.at[pl.ds(base + r * CH, CH), :])
        plsc.subcore_barrier()

        def _block(b, carry):
            pltpu.sync_copy(src_hbm.at[pl.ds(srow0 + b * IB, IB), :], src_v)
            pltpu.sync_copy(dst_hbm.at[pl.ds(drow0 + b * IB, IB), :], dst_v)
            gd = [None, None]
            sd = [None, None]
            gd[0] = pltpu.async_copy(h_hbm.at[src_v.at[0]], bufs[0], sgs[0])
            for i in range(IB):
                p, q = i % 2, (i + 1) % 2
                if i + 1 < IB:
                    if sd[q] is not None:
                        sd[q].wait()
                    gd[q] = pltpu.async_copy(h_hbm.at[src_v.at[i + 1]],
                                             bufs[q], sgs[q])
                gd[p].wait()
                sd[p] = pltpu.async_copy(bufs[p], acc_sh.at[dst_v.at[i]],
                                         sss[p], add=True)
            sd[0].wait()
            sd[1].wait()
            return carry

        lax.fori_loop(0, n_chunks // IB, _block, 0)
        plsc.subcore_barrier()
        obase = c * NPAD + base
        for r in range(ROWS // CH):
            pltpu.sync_copy(acc_sh.at[pl.ds(base + r * CH, CH), :], buf_a)
            pltpu.sync_copy(buf_a, out_hbm.at[pl.ds(obase + r * CH, CH), :])

    return _msg


_msg1 = _make_msg(HID2 // 2, edge_split=False)
_msg2 = _make_msg(OUT_DIM, edge_split=True)


def _tc1_body(x_ref, w1_ref, deg_ref, out_ref):
    h = jnp.dot(x_ref[...], w1_ref[...], preferred_element_type=jnp.float32)
    dinv = lax.rsqrt(deg_ref[...] + 1.0)
    hs = h * dinv
    out_ref[0:N, :] = hs[:, 0:HID2 // 2]
    out_ref[N:2 * N, :] = hs[:, HID2 // 2:HID2]


def _tc2_body(acc_ref, h_ref, deg_ref, g_ref, be_ref, a_ref, w2_ref, b1_ref,
              out_ref):
    F = HID2 // 2
    dinv = lax.rsqrt(deg_ref[...] + 1.0)
    a = a_ref[0, 0]
    w2 = w2_ref[...]
    h2 = None
    for k in range(2):
        z = (acc_ref[k * NPAD:k * NPAD + N, :] + h_ref[k * N:(k + 1) * N, :])
        z = z * dinv + b1_ref[k:k + 1, :]
        mean = jnp.sum(z, axis=0, keepdims=True) * (1.0 / N)
        d = z - mean
        var = jnp.sum(d * d, axis=0, keepdims=True) * (1.0 / N)
        zn = d * lax.rsqrt(var + 1e-5) * g_ref[k:k + 1, :] + be_ref[k:k + 1, :]
        zp = jnp.where(zn > 0, zn, a * zn)
        part = jnp.dot(zp, w2[k], preferred_element_type=jnp.float32)
        h2 = part if h2 is None else h2 + part
    out_ref[...] = h2 * dinv


def _tc3_body(acc_ref, h_ref, deg_ref, b2_ref, out_ref):
    dinv = lax.rsqrt(deg_ref[...] + 1.0)
    total = acc_ref[0:N, :] + acc_ref[NPAD:NPAD + N, :] + h_ref[...]
    out_ref[...] = total * dinv + b2_ref[...]


_tc1 = pl.pallas_call(
    _tc1_body,
    out_shape=jax.ShapeDtypeStruct((2 * N, HID2 // 2), jnp.float32),
)
_tc2 = pl.pallas_call(
    _tc2_body,
    out_shape=jax.ShapeDtypeStruct((N, OUT_DIM), jnp.float32),
)
_tc3 = pl.pallas_call(
    _tc3_body,
    out_shape=jax.ShapeDtypeStruct((N, OUT_DIM), jnp.float32),
)


def kernel(x, edge_index, W1, b1, gamma, beta, a, W2, b2):
    src = edge_index[0].astype(jnp.int32)
    dst = edge_index[1].astype(jnp.int32)
    P = EPAD - E
    pad = jnp.arange(P, dtype=jnp.int32)
    src_p = jnp.concatenate([src, pad % N]).reshape(NS * CHUNKS, CH)
    dst_p = jnp.concatenate([dst, N + pad % (NPAD - N)]).reshape(NS * CHUNKS, CH)
    src_both = jnp.concatenate([src_p, src_p + N]).reshape(NC * NS * CHUNKS, CH)

    deg = _deg_call(dst_p)[:N].reshape(N, 1)
    h1s = _tc1(x, W1, deg)
    acc1 = _msg1(h1s, src_both, dst_p)
    h2s = _tc2(acc1, h1s, deg, gamma.reshape(2, HID2 // 2),
               beta.reshape(2, HID2 // 2), a.reshape(1, 1),
               W2.reshape(2, HID2 // 2, OUT_DIM), b1.reshape(2, HID2 // 2))
    acc2 = _msg2(h2s, src_p, dst_p)
    return _tc3(acc2, h2s, deg, b2.reshape(1, OUT_DIM))


# speedup vs baseline: 27.4064x; 1.0302x over previous
import functools

import jax
import jax.numpy as jnp
from jax import lax
from jax.experimental import pallas as pl
from jax.experimental.pallas import tpu as pltpu
from jax.experimental.pallas import tpu_sc as plsc

N = 10000
E = 320000
IN_DIM = 128
HID2 = 256
OUT_DIM = 128

NC = 2
NS = 16
L = 16

NPAD = 10240
ROWS = NPAD // NS
CH = 128
CHUNKS = 160
IB = 16
EPAD = NS * CHUNKS * CH


def _sc_mesh():
    return plsc.VectorSubcoreMesh(core_axis_name="c", subcore_axis_name="s")


@functools.partial(
    pl.kernel,
    out_type=jax.ShapeDtypeStruct((NC * NPAD,), jnp.float32),
    mesh=_sc_mesh(),
    scratch_types=[
        pltpu.VMEM((IB, CH), jnp.int32),
        pltpu.VMEM((CH,), jnp.float32),
        pltpu.VMEM((ROWS,), jnp.float32),
        pltpu.VMEM_SHARED((NPAD,), jnp.float32),
        pltpu.SemaphoreType.DMA,
        pltpu.SemaphoreType.DMA,
    ],
)
def _deg_call(dst_hbm, out_hbm, dst_v, ones_v, zb, deg_sh, sem0, sem1):
    c = lax.axis_index("c")
    s = lax.axis_index("s")
    base = s * ROWS
    for i in range(CH // L):
        ones_v[pl.ds(i * L, L)] = jnp.ones((L,), jnp.float32)
    for i in range(ROWS // L):
        zb[pl.ds(i * L, L)] = jnp.zeros((L,), jnp.float32)
    pltpu.sync_copy(zb, deg_sh.at[pl.ds(base, ROWS)])
    plsc.subcore_barrier()
    sems = (sem0, sem1)

    def _block(b, carry):
        pltpu.sync_copy(dst_hbm.at[pl.ds(s * CHUNKS + b * IB, IB), :], dst_v)
        descs = [None, None]
        for i in range(IB):
            p = i % 2
            if descs[p] is not None:
                descs[p].wait()
            descs[p] = pltpu.async_copy(ones_v, deg_sh.at[dst_v.at[i]],
                                        sems[p], add=True)
        descs[0].wait()
        descs[1].wait()
        return carry

    lax.fori_loop(0, CHUNKS // IB, _block, 0)
    plsc.subcore_barrier()
    pltpu.sync_copy(deg_sh.at[pl.ds(base, ROWS)], zb)
    pltpu.sync_copy(zb, out_hbm.at[pl.ds(c * NPAD + base, ROWS)])


def _make_msg(F2, edge_split, ib=IB):
    @functools.partial(
        pl.kernel,
        out_type=jax.ShapeDtypeStruct((NC * NPAD, F2), jnp.float32),
        mesh=_sc_mesh(),
        scratch_types=[
            pltpu.VMEM((ib, CH), jnp.int32),
            pltpu.VMEM((ib, CH), jnp.int32),
            pltpu.VMEM((CH, F2), jnp.float32),
            pltpu.VMEM((CH, F2), jnp.float32),
            pltpu.VMEM_SHARED((NPAD, F2), jnp.float32),
            pltpu.SemaphoreType.DMA,
            pltpu.SemaphoreType.DMA,
            pltpu.SemaphoreType.DMA,
            pltpu.SemaphoreType.DMA,
        ],
    )
    def _msg(h_hbm, src_hbm, dst_hbm, out_hbm, src_v, dst_v, buf_a, buf_b,
             acc_sh, sg0, sg1, ss0, ss1):
        c = lax.axis_index("c")
        s = lax.axis_index("s")
        base = s * ROWS
        if edge_split:
            n_chunks = CHUNKS // NC
            srow0 = drow0 = (c * NS + s) * n_chunks
        else:
            n_chunks = CHUNKS
            srow0 = (c * NS + s) * CHUNKS
            drow0 = s * CHUNKS
        bufs = (buf_a, buf_b)
        sgs = (sg0, sg1)
        sss = (ss0, ss1)

        def _zrow(i, carry):
            for f in range(F2 // L):
                buf_a[i, pl.ds(f * L, L)] = jnp.zeros((L,), jnp.float32)
            return carry

        lax.fori_loop(0, CH, _zrow, 0)
        for r in range(ROWS // CH):
            pltpu.sync_copy(buf_a, acc_sh.at[pl.ds(base + r * CH, CH), :])
        plsc.subcore_barrier()

        def _block(b, carry):
            pltpu.sync_copy(src_hbm.at[pl.ds(srow0 + b * ib, ib), :], src_v)
            pltpu.sync_copy(dst_hbm.at[pl.ds(drow0 + b * ib, ib), :], dst_v)
            gd = [None, None]
            sd = [None, None]
            gd[0] = pltpu.async_copy(h_hbm.at[src_v.at[0]], bufs[0], sgs[0])
            for i in range(ib):
                p, q = i % 2, (i + 1) % 2
                if i + 1 < ib:
                    if sd[q] is not None:
                        sd[q].wait()
                    gd[q] = pltpu.async_copy(h_hbm.at[src_v.at[i + 1]],
                                             bufs[q], sgs[q])
                gd[p].wait()
                sd[p] = pltpu.async_copy(bufs[p], acc_sh.at[dst_v.at[i]],
                                         sss[p], add=True)
            sd[0].wait()
            sd[1].wait()
            return carry

        lax.fori_loop(0, n_chunks // ib, _block, 0)
        plsc.subcore_barrier()
        obase = c * NPAD + base
        for r in range(ROWS // CH):
            pltpu.sync_copy(acc_sh.at[pl.ds(base + r * CH, CH), :], buf_a)
            pltpu.sync_copy(buf_a, out_hbm.at[pl.ds(obase + r * CH, CH), :])

    return _msg


_msg1 = _make_msg(HID2 // 2, edge_split=False, ib=32)
_msg2 = _make_msg(OUT_DIM, edge_split=True)


def _tc1_body(x_ref, w1_ref, deg_ref, out_ref):
    h = jnp.dot(x_ref[...], w1_ref[...], preferred_element_type=jnp.float32)
    dinv = lax.rsqrt(deg_ref[...] + 1.0)
    hs = h * dinv
    out_ref[0:N, :] = hs[:, 0:HID2 // 2]
    out_ref[N:2 * N, :] = hs[:, HID2 // 2:HID2]


def _tc2_body(acc_ref, h_ref, deg_ref, g_ref, be_ref, a_ref, w2_ref, b1_ref,
              out_ref):
    F = HID2 // 2
    dinv = lax.rsqrt(deg_ref[...] + 1.0)
    a = a_ref[0, 0]
    w2 = w2_ref[...]
    h2 = None
    for k in range(2):
        z = (acc_ref[k * NPAD:k * NPAD + N, :] + h_ref[k * N:(k + 1) * N, :])
        z = z * dinv + b1_ref[k:k + 1, :]
        mean = jnp.sum(z, axis=0, keepdims=True) * (1.0 / N)
        d = z - mean
        var = jnp.sum(d * d, axis=0, keepdims=True) * (1.0 / N)
        zn = d * lax.rsqrt(var + 1e-5) * g_ref[k:k + 1, :] + be_ref[k:k + 1, :]
        zp = jnp.where(zn > 0, zn, a * zn)
        part = jnp.dot(zp, w2[k], preferred_element_type=jnp.float32)
        h2 = part if h2 is None else h2 + part
    out_ref[...] = h2 * dinv


def _tc3_body(acc_ref, h_ref, deg_ref, b2_ref, out_ref):
    dinv = lax.rsqrt(deg_ref[...] + 1.0)
    total = acc_ref[0:N, :] + acc_ref[NPAD:NPAD + N, :] + h_ref[...]
    out_ref[...] = total * dinv + b2_ref[...]


_tc1 = pl.pallas_call(
    _tc1_body,
    out_shape=jax.ShapeDtypeStruct((2 * N, HID2 // 2), jnp.float32),
)
_tc2 = pl.pallas_call(
    _tc2_body,
    out_shape=jax.ShapeDtypeStruct((N, OUT_DIM), jnp.float32),
)
_tc3 = pl.pallas_call(
    _tc3_body,
    out_shape=jax.ShapeDtypeStruct((N, OUT_DIM), jnp.float32),
)


def kernel(x, edge_index, W1, b1, gamma, beta, a, W2, b2):
    src = edge_index[0].astype(jnp.int32)
    dst = edge_index[1].astype(jnp.int32)
    P = EPAD - E
    pad = jnp.arange(P, dtype=jnp.int32)
    src_p = jnp.concatenate([src, pad % N]).reshape(NS * CHUNKS, CH)
    dst_p = jnp.concatenate([dst, N + pad % (NPAD - N)]).reshape(NS * CHUNKS, CH)
    src_both = jnp.concatenate([src_p, src_p + N]).reshape(NC * NS * CHUNKS, CH)

    deg = _deg_call(dst_p)[:N].reshape(N, 1)
    h1s = _tc1(x, W1, deg)
    acc1 = _msg1(h1s, src_both, dst_p)
    h2s = _tc2(acc1, h1s, deg, gamma.reshape(2, HID2 // 2),
               beta.reshape(2, HID2 // 2), a.reshape(1, 1),
               W2.reshape(2, HID2 // 2, OUT_DIM), b1.reshape(2, HID2 // 2))
    acc2 = _msg2(h2s, src_p, dst_p)
    return _tc3(acc2, h2s, deg, b2.reshape(1, OUT_DIM))


# speedup vs baseline: 35.0760x; 1.2798x over previous
import functools

import jax
import jax.numpy as jnp
from jax import lax
from jax.experimental import pallas as pl
from jax.experimental.pallas import tpu as pltpu
from jax.experimental.pallas import tpu_sc as plsc

N = 10000
E = 320000
IN_DIM = 128
HID2 = 256
OUT_DIM = 128
F2 = 128

NC = 2
NS = 16
L = 16

NPAD = 10240
ROWS = NPAD // NS
CH = 128
CHUNKS = 160
TCH = CHUNKS // NC
IB = 16
EPAD = NS * CHUNKS * CH


def _sc_mesh():
    return plsc.VectorSubcoreMesh(core_axis_name="c", subcore_axis_name="s")


@functools.partial(
    pl.kernel,
    out_type=jax.ShapeDtypeStruct((NC * NPAD,), jnp.float32),
    mesh=_sc_mesh(),
    scratch_types=[
        pltpu.VMEM((IB, CH), jnp.int32),
        pltpu.VMEM((CH,), jnp.float32),
        pltpu.VMEM((ROWS,), jnp.float32),
        pltpu.VMEM_SHARED((NPAD,), jnp.float32),
        pltpu.SemaphoreType.DMA,
        pltpu.SemaphoreType.DMA,
    ],
)
def _deg_call(dst_hbm, out_hbm, dst_v, ones_v, zb, deg_sh, sem0, sem1):
    c = lax.axis_index("c")
    s = lax.axis_index("s")
    base = s * ROWS
    for i in range(CH // L):
        ones_v[pl.ds(i * L, L)] = jnp.ones((L,), jnp.float32)
    for i in range(ROWS // L):
        zb[pl.ds(i * L, L)] = jnp.zeros((L,), jnp.float32)
    pltpu.sync_copy(zb, deg_sh.at[pl.ds(base, ROWS)])
    plsc.subcore_barrier()
    sems = (sem0, sem1)

    def _block(b, carry):
        pltpu.sync_copy(dst_hbm.at[pl.ds(s * CHUNKS + b * IB, IB), :], dst_v)
        descs = [None, None]
        for i in range(IB):
            p = i % 2
            if descs[p] is not None:
                descs[p].wait()
            descs[p] = pltpu.async_copy(ones_v, deg_sh.at[dst_v.at[i]],
                                        sems[p], add=True)
        descs[0].wait()
        descs[1].wait()
        return carry

    lax.fori_loop(0, CHUNKS // IB, _block, 0)
    plsc.subcore_barrier()
    pltpu.sync_copy(deg_sh.at[pl.ds(base, ROWS)], zb)
    pltpu.sync_copy(zb, out_hbm.at[pl.ds(c * NPAD + base, ROWS)])


@functools.partial(
    pl.kernel,
    out_type=jax.ShapeDtypeStruct((NC * NPAD, F2), jnp.float32),
    mesh=_sc_mesh(),
    scratch_types=[
        pltpu.VMEM((IB, CH), jnp.int32),
        pltpu.VMEM((IB, CH), jnp.int32),
        pltpu.VMEM((CH, F2), jnp.float32),
        pltpu.VMEM((CH, F2), jnp.float32),
        pltpu.VMEM_SHARED((NPAD, F2), jnp.float32),
        pltpu.SemaphoreType.DMA,
        pltpu.SemaphoreType.DMA,
        pltpu.SemaphoreType.DMA,
        pltpu.SemaphoreType.DMA,
    ],
)
def _msg_call(h_hbm, src_hbm, dst_hbm, out_hbm, src_v, dst_v, buf_a, buf_b,
              acc_sh, sg0, sg1, ss0, ss1):
    c = lax.axis_index("c")
    s = lax.axis_index("s")
    base = s * ROWS
    row0 = (c * NS + s) * TCH
    bufs = (buf_a, buf_b)
    sgs = (sg0, sg1)
    sss = (ss0, ss1)

    def _zrow(i, carry):
        for f in range(F2 // L):
            buf_a[i, pl.ds(f * L, L)] = jnp.zeros((L,), jnp.float32)
        return carry

    lax.fori_loop(0, CH, _zrow, 0)
    for r in range(ROWS // CH):
        pltpu.sync_copy(buf_a, acc_sh.at[pl.ds(base + r * CH, CH), :])
    plsc.subcore_barrier()

    def _block(b, carry):
        pltpu.sync_copy(src_hbm.at[pl.ds(row0 + b * IB, IB), :], src_v)
        pltpu.sync_copy(dst_hbm.at[pl.ds(row0 + b * IB, IB), :], dst_v)
        gd = [None, None]
        sd = [None, None]
        gd[0] = pltpu.async_copy(h_hbm.at[src_v.at[0]], bufs[0], sgs[0])
        for i in range(IB):
            p, q = i % 2, (i + 1) % 2
            if i + 1 < IB:
                if sd[q] is not None:
                    sd[q].wait()
                gd[q] = pltpu.async_copy(h_hbm.at[src_v.at[i + 1]],
                                         bufs[q], sgs[q])
            gd[p].wait()
            sd[p] = pltpu.async_copy(bufs[p], acc_sh.at[dst_v.at[i]],
                                     sss[p], add=True)
        sd[0].wait()
        sd[1].wait()
        return carry

    lax.fori_loop(0, TCH // IB, _block, 0)
    plsc.subcore_barrier()
    obase = c * NPAD + base
    for r in range(ROWS // CH):
        pltpu.sync_copy(acc_sh.at[pl.ds(base + r * CH, CH), :], buf_a)
        pltpu.sync_copy(buf_a, out_hbm.at[pl.ds(obase + r * CH, CH), :])


def _tc1_body(x_ref, deg_ref, out_ref):
    dinv = lax.rsqrt(deg_ref[...] + 1.0)
    out_ref[...] = x_ref[...] * dinv


def _tc2_body(acc_ref, xs_ref, deg_ref, w1_ref, b1_ref, g_ref, be_ref, a_ref,
              w2_ref, out_ref):
    dinv = lax.rsqrt(deg_ref[...] + 1.0)
    agg = (acc_ref[0:N, :] + acc_ref[NPAD:NPAD + N, :] + xs_ref[...]) * dinv
    z = jnp.dot(agg, w1_ref[...], preferred_element_type=jnp.float32)
    z = z + b1_ref[...]
    mean = jnp.sum(z, axis=0, keepdims=True) * (1.0 / N)
    d = z - mean
    var = jnp.sum(d * d, axis=0, keepdims=True) * (1.0 / N)
    zn = d * lax.rsqrt(var + 1e-5) * g_ref[...] + be_ref[...]
    zp = jnp.where(zn > 0, zn, a_ref[0, 0] * zn)
    h2 = jnp.dot(zp, w2_ref[...], preferred_element_type=jnp.float32)
    out_ref[...] = h2 * dinv


def _tc3_body(acc_ref, h_ref, deg_ref, b2_ref, out_ref):
    dinv = lax.rsqrt(deg_ref[...] + 1.0)
    total = acc_ref[0:N, :] + acc_ref[NPAD:NPAD + N, :] + h_ref[...]
    out_ref[...] = total * dinv + b2_ref[...]


_tc1 = pl.pallas_call(
    _tc1_body,
    out_shape=jax.ShapeDtypeStruct((N, IN_DIM), jnp.float32),
)
_tc2 = pl.pallas_call(
    _tc2_body,
    out_shape=jax.ShapeDtypeStruct((N, OUT_DIM), jnp.float32),
)
_tc3 = pl.pallas_call(
    _tc3_body,
    out_shape=jax.ShapeDtypeStruct((N, OUT_DIM), jnp.float32),
)


def kernel(x, edge_index, W1, b1, gamma, beta, a, W2, b2):
    src = edge_index[0].astype(jnp.int32)
    dst = edge_index[1].astype(jnp.int32)
    P = EPAD - E
    pad = jnp.arange(P, dtype=jnp.int32)
    src_p = jnp.concatenate([src, pad % N]).reshape(NS * CHUNKS, CH)
    dst_p = jnp.concatenate([dst, N + pad % (NPAD - N)]).reshape(NS * CHUNKS, CH)

    deg = _deg_call(dst_p)[:N].reshape(N, 1)
    xs = _tc1(x, deg)
    acc1 = _msg_call(xs, src_p, dst_p)
    h2s = _tc2(acc1, xs, deg, W1, b1.reshape(1, HID2), gamma.reshape(1, HID2),
               beta.reshape(1, HID2), a.reshape(1, 1), W2)
    acc2 = _msg_call(h2s, src_p, dst_p)
    return _tc3(acc2, h2s, deg, b2.reshape(1, OUT_DIM))


# speedup vs baseline: 36.8544x; 1.0507x over previous
import functools

import jax
import jax.numpy as jnp
from jax import lax
from jax.experimental import pallas as pl
from jax.experimental.pallas import tpu as pltpu
from jax.experimental.pallas import tpu_sc as plsc

N = 10000
E = 320000
IN_DIM = 128
HID2 = 256
OUT_DIM = 128
F2 = 128

NC = 2
NS = 16
L = 16

NPAD = 10240
ROWS = NPAD // NS
CH = 128
CHUNKS = 160
TCH = CHUNKS // NC
IB = 16
IBM = 40
EPAD = NS * CHUNKS * CH


def _sc_mesh():
    return plsc.VectorSubcoreMesh(core_axis_name="c", subcore_axis_name="s")


@functools.partial(
    pl.kernel,
    out_type=jax.ShapeDtypeStruct((NC * NPAD,), jnp.float32),
    mesh=_sc_mesh(),
    scratch_types=[
        pltpu.VMEM((IB, CH), jnp.int32),
        pltpu.VMEM((CH,), jnp.float32),
        pltpu.VMEM((ROWS,), jnp.float32),
        pltpu.VMEM_SHARED((NPAD,), jnp.float32),
        pltpu.SemaphoreType.DMA,
        pltpu.SemaphoreType.DMA,
    ],
)
def _deg_call(dst_hbm, out_hbm, dst_v, ones_v, zb, deg_sh, sem0, sem1):
    c = lax.axis_index("c")
    s = lax.axis_index("s")
    base = s * ROWS
    for i in range(CH // L):
        ones_v[pl.ds(i * L, L)] = jnp.ones((L,), jnp.float32)
    for i in range(ROWS // L):
        zb[pl.ds(i * L, L)] = jnp.zeros((L,), jnp.float32)
    pltpu.sync_copy(zb, deg_sh.at[pl.ds(base, ROWS)])
    plsc.subcore_barrier()
    sems = (sem0, sem1)

    def _block(b, carry):
        pltpu.sync_copy(dst_hbm.at[pl.ds(s * CHUNKS + b * IB, IB), :], dst_v)
        descs = [None, None]
        for i in range(IB):
            p = i % 2
            if descs[p] is not None:
                descs[p].wait()
            descs[p] = pltpu.async_copy(ones_v, deg_sh.at[dst_v.at[i]],
                                        sems[p], add=True)
        descs[0].wait()
        descs[1].wait()
        return carry

    lax.fori_loop(0, CHUNKS // IB, _block, 0)
    plsc.subcore_barrier()
    pltpu.sync_copy(deg_sh.at[pl.ds(base, ROWS)], zb)
    pltpu.sync_copy(zb, out_hbm.at[pl.ds(c * NPAD + base, ROWS)])


@functools.partial(
    pl.kernel,
    out_type=jax.ShapeDtypeStruct((NC * NPAD, F2), jnp.float32),
    mesh=_sc_mesh(),
    scratch_types=[
        pltpu.VMEM((IBM, CH), jnp.int32),
        pltpu.VMEM((IBM, CH), jnp.int32),
        pltpu.VMEM((CH, F2), jnp.float32),
        pltpu.VMEM((CH, F2), jnp.float32),
        pltpu.VMEM_SHARED((NPAD, F2), jnp.float32),
        pltpu.SemaphoreType.DMA,
        pltpu.SemaphoreType.DMA,
        pltpu.SemaphoreType.DMA,
        pltpu.SemaphoreType.DMA,
    ],
)
def _msg_call(h_hbm, src_hbm, dst_hbm, out_hbm, src_v, dst_v, buf_a, buf_b,
              acc_sh, sg0, sg1, ss0, ss1):
    c = lax.axis_index("c")
    s = lax.axis_index("s")
    base = s * ROWS
    row0 = (c * NS + s) * TCH
    bufs = (buf_a, buf_b)
    sgs = (sg0, sg1)
    sss = (ss0, ss1)

    def _zrow(i, carry):
        for f in range(F2 // L):
            buf_a[i, pl.ds(f * L, L)] = jnp.zeros((L,), jnp.float32)
        return carry

    lax.fori_loop(0, CH, _zrow, 0)
    for r in range(ROWS // CH):
        pltpu.sync_copy(buf_a, acc_sh.at[pl.ds(base + r * CH, CH), :])
    plsc.subcore_barrier()

    def _block(b, carry):
        pltpu.sync_copy(src_hbm.at[pl.ds(row0 + b * IBM, IBM), :], src_v)
        pltpu.sync_copy(dst_hbm.at[pl.ds(row0 + b * IBM, IBM), :], dst_v)
        gd = [None, None]
        sd = [None, None]
        gd[0] = pltpu.async_copy(h_hbm.at[src_v.at[0]], bufs[0], sgs[0])
        for i in range(IBM):
            p, q = i % 2, (i + 1) % 2
            if i + 1 < IBM:
                if sd[q] is not None:
                    sd[q].wait()
                gd[q] = pltpu.async_copy(h_hbm.at[src_v.at[i + 1]],
                                         bufs[q], sgs[q])
            gd[p].wait()
            sd[p] = pltpu.async_copy(bufs[p], acc_sh.at[dst_v.at[i]],
                                     sss[p], add=True)
        sd[0].wait()
        sd[1].wait()
        return carry

    lax.fori_loop(0, TCH // IBM, _block, 0)
    plsc.subcore_barrier()
    obase = c * NPAD + base
    for r in range(ROWS // CH):
        pltpu.sync_copy(acc_sh.at[pl.ds(base + r * CH, CH), :], buf_a)
        pltpu.sync_copy(buf_a, out_hbm.at[pl.ds(obase + r * CH, CH), :])


def _tc1_body(x_ref, deg_ref, out_ref):
    dinv = lax.rsqrt(deg_ref[...] + 1.0)
    out_ref[...] = x_ref[...] * dinv


def _tc2_body(acc_ref, xs_ref, deg_ref, w1_ref, b1_ref, g_ref, be_ref, a_ref,
              w2_ref, out_ref):
    dinv = lax.rsqrt(deg_ref[...] + 1.0)
    agg = (acc_ref[0:N, :] + acc_ref[NPAD:NPAD + N, :] + xs_ref[...]) * dinv
    z = jnp.dot(agg, w1_ref[...], preferred_element_type=jnp.float32)
    z = z + b1_ref[...]
    mean = jnp.sum(z, axis=0, keepdims=True) * (1.0 / N)
    d = z - mean
    var = jnp.sum(d * d, axis=0, keepdims=True) * (1.0 / N)
    zn = d * lax.rsqrt(var + 1e-5) * g_ref[...] + be_ref[...]
    zp = jnp.where(zn > 0, zn, a_ref[0, 0] * zn)
    h2 = jnp.dot(zp, w2_ref[...], preferred_element_type=jnp.float32)
    out_ref[...] = h2 * dinv


def _tc3_body(acc_ref, h_ref, deg_ref, b2_ref, out_ref):
    dinv = lax.rsqrt(deg_ref[...] + 1.0)
    total = acc_ref[0:N, :] + acc_ref[NPAD:NPAD + N, :] + h_ref[...]
    out_ref[...] = total * dinv + b2_ref[...]


_tc1 = pl.pallas_call(
    _tc1_body,
    out_shape=jax.ShapeDtypeStruct((N, IN_DIM), jnp.float32),
)
_tc2 = pl.pallas_call(
    _tc2_body,
    out_shape=jax.ShapeDtypeStruct((N, OUT_DIM), jnp.float32),
)
_tc3 = pl.pallas_call(
    _tc3_body,
    out_shape=jax.ShapeDtypeStruct((N, OUT_DIM), jnp.float32),
)


def kernel(x, edge_index, W1, b1, gamma, beta, a, W2, b2):
    src = edge_index[0].astype(jnp.int32)
    dst = edge_index[1].astype(jnp.int32)
    P = EPAD - E
    pad = jnp.arange(P, dtype=jnp.int32)
    src_p = jnp.concatenate([src, pad % N]).reshape(NS * CHUNKS, CH)
    dst_p = jnp.concatenate([dst, N + pad % (NPAD - N)]).reshape(NS * CHUNKS, CH)

    deg = _deg_call(dst_p)[:N].reshape(N, 1)
    xs = _tc1(x, deg)
    acc1 = _msg_call(xs, src_p, dst_p)
    h2s = _tc2(acc1, xs, deg, W1, b1.reshape(1, HID2), gamma.reshape(1, HID2),
               beta.reshape(1, HID2), a.reshape(1, 1), W2)
    acc2 = _msg_call(h2s, src_p, dst_p)
    return _tc3(acc2, h2s, deg, b2.reshape(1, OUT_DIM))


# speedup vs baseline: 37.0455x; 1.0052x over previous
import functools

import jax
import jax.numpy as jnp
from jax import lax
from jax.experimental import pallas as pl
from jax.experimental.pallas import tpu as pltpu
from jax.experimental.pallas import tpu_sc as plsc

N = 10000
E = 320000
IN_DIM = 128
HID2 = 256
OUT_DIM = 128
F2 = 128

NC = 2
NS = 16
L = 16

NPAD = 10240
ROWS = NPAD // NS
CH = 128
CHUNKS = 160
TCH = CHUNKS // NC
IB = 16
IBM = 40
EPAD = NS * CHUNKS * CH


def _sc_mesh():
    return plsc.VectorSubcoreMesh(core_axis_name="c", subcore_axis_name="s")


@functools.partial(
    pl.kernel,
    out_type=jax.ShapeDtypeStruct((NC * NPAD,), jnp.float32),
    mesh=_sc_mesh(),
    scratch_types=[
        pltpu.VMEM((IB, CH), jnp.int32),
        pltpu.VMEM((CH,), jnp.float32),
        pltpu.VMEM((ROWS,), jnp.float32),
        pltpu.VMEM_SHARED((NPAD,), jnp.float32),
        pltpu.SemaphoreType.DMA,
        pltpu.SemaphoreType.DMA,
    ],
)
def _deg_call(dst_hbm, out_hbm, dst_v, ones_v, zb, deg_sh, sem0, sem1):
    c = lax.axis_index("c")
    s = lax.axis_index("s")
    base = s * ROWS
    for i in range(CH // L):
        ones_v[pl.ds(i * L, L)] = jnp.ones((L,), jnp.float32)
    for i in range(ROWS // L):
        zb[pl.ds(i * L, L)] = jnp.zeros((L,), jnp.float32)
    pltpu.sync_copy(zb, deg_sh.at[pl.ds(base, ROWS)])
    plsc.subcore_barrier()
    sems = (sem0, sem1)

    row0 = (c * NS + s) * TCH

    def _block(b, carry):
        pltpu.sync_copy(dst_hbm.at[pl.ds(row0 + b * IB, IB), :], dst_v)
        descs = [None, None]
        for i in range(IB):
            p = i % 2
            if descs[p] is not None:
                descs[p].wait()
            descs[p] = pltpu.async_copy(ones_v, deg_sh.at[dst_v.at[i]],
                                        sems[p], add=True)
        descs[0].wait()
        descs[1].wait()
        return carry

    lax.fori_loop(0, TCH // IB, _block, 0)
    plsc.subcore_barrier()
    pltpu.sync_copy(deg_sh.at[pl.ds(base, ROWS)], zb)
    pltpu.sync_copy(zb, out_hbm.at[pl.ds(c * NPAD + base, ROWS)])


@functools.partial(
    pl.kernel,
    out_type=jax.ShapeDtypeStruct((NC * NPAD, F2), jnp.float32),
    mesh=_sc_mesh(),
    scratch_types=[
        pltpu.VMEM((IBM, CH), jnp.int32),
        pltpu.VMEM((IBM, CH), jnp.int32),
        pltpu.VMEM((CH, F2), jnp.float32),
        pltpu.VMEM((CH, F2), jnp.float32),
        pltpu.VMEM_SHARED((NPAD, F2), jnp.float32),
        pltpu.SemaphoreType.DMA,
        pltpu.SemaphoreType.DMA,
        pltpu.SemaphoreType.DMA,
        pltpu.SemaphoreType.DMA,
    ],
)
def _msg_call(h_hbm, src_hbm, dst_hbm, out_hbm, src_v, dst_v, buf_a, buf_b,
              acc_sh, sg0, sg1, ss0, ss1):
    c = lax.axis_index("c")
    s = lax.axis_index("s")
    base = s * ROWS
    row0 = (c * NS + s) * TCH
    bufs = (buf_a, buf_b)
    sgs = (sg0, sg1)
    sss = (ss0, ss1)

    def _zrow(i, carry):
        for f in range(F2 // L):
            buf_a[i, pl.ds(f * L, L)] = jnp.zeros((L,), jnp.float32)
        return carry

    lax.fori_loop(0, CH, _zrow, 0)
    for r in range(ROWS // CH):
        pltpu.sync_copy(buf_a, acc_sh.at[pl.ds(base + r * CH, CH), :])
    plsc.subcore_barrier()

    def _block(b, carry):
        pltpu.sync_copy(src_hbm.at[pl.ds(row0 + b * IBM, IBM), :], src_v)
        pltpu.sync_copy(dst_hbm.at[pl.ds(row0 + b * IBM, IBM), :], dst_v)
        gd = [None, None]
        sd = [None, None]
        gd[0] = pltpu.async_copy(h_hbm.at[src_v.at[0]], bufs[0], sgs[0])
        for i in range(IBM):
            p, q = i % 2, (i + 1) % 2
            if i + 1 < IBM:
                if sd[q] is not None:
                    sd[q].wait()
                gd[q] = pltpu.async_copy(h_hbm.at[src_v.at[i + 1]],
                                         bufs[q], sgs[q])
            gd[p].wait()
            sd[p] = pltpu.async_copy(bufs[p], acc_sh.at[dst_v.at[i]],
                                     sss[p], add=True)
        sd[0].wait()
        sd[1].wait()
        return carry

    lax.fori_loop(0, TCH // IBM, _block, 0)
    plsc.subcore_barrier()
    obase = c * NPAD + base
    for r in range(ROWS // CH):
        pltpu.sync_copy(acc_sh.at[pl.ds(base + r * CH, CH), :], buf_a)
        pltpu.sync_copy(buf_a, out_hbm.at[pl.ds(obase + r * CH, CH), :])


def _dinv(deg_ref):
    return lax.rsqrt(deg_ref[0:N, :] + deg_ref[NPAD:NPAD + N, :] + 1.0)


def _tc1_body(x_ref, deg_ref, out_ref):
    dinv = _dinv(deg_ref)
    out_ref[...] = x_ref[...] * dinv


def _tc2_body(acc_ref, xs_ref, deg_ref, w1_ref, b1_ref, g_ref, be_ref, a_ref,
              w2_ref, out_ref):
    dinv = _dinv(deg_ref)
    agg = (acc_ref[0:N, :] + acc_ref[NPAD:NPAD + N, :] + xs_ref[...]) * dinv
    z = jnp.dot(agg, w1_ref[...], preferred_element_type=jnp.float32)
    z = z + b1_ref[...]
    mean = jnp.sum(z, axis=0, keepdims=True) * (1.0 / N)
    d = z - mean
    var = jnp.sum(d * d, axis=0, keepdims=True) * (1.0 / N)
    zn = d * lax.rsqrt(var + 1e-5) * g_ref[...] + be_ref[...]
    zp = jnp.where(zn > 0, zn, a_ref[0, 0] * zn)
    h2 = jnp.dot(zp, w2_ref[...], preferred_element_type=jnp.float32)
    out_ref[...] = h2 * dinv


def _tc3_body(acc_ref, h_ref, deg_ref, b2_ref, out_ref):
    dinv = _dinv(deg_ref)
    total = acc_ref[0:N, :] + acc_ref[NPAD:NPAD + N, :] + h_ref[...]
    out_ref[...] = total * dinv + b2_ref[...]


_tc1 = pl.pallas_call(
    _tc1_body,
    out_shape=jax.ShapeDtypeStruct((N, IN_DIM), jnp.float32),
)
_tc2 = pl.pallas_call(
    _tc2_body,
    out_shape=jax.ShapeDtypeStruct((N, OUT_DIM), jnp.float32),
)
_tc3 = pl.pallas_call(
    _tc3_body,
    out_shape=jax.ShapeDtypeStruct((N, OUT_DIM), jnp.float32),
)


def kernel(x, edge_index, W1, b1, gamma, beta, a, W2, b2):
    src = edge_index[0].astype(jnp.int32)
    dst = edge_index[1].astype(jnp.int32)
    P = EPAD - E
    pad = jnp.arange(P, dtype=jnp.int32)
    src_p = jnp.concatenate([src, pad % N]).reshape(NS * CHUNKS, CH)
    dst_p = jnp.concatenate([dst, N + pad % (NPAD - N)]).reshape(NS * CHUNKS, CH)

    deg = _deg_call(dst_p).reshape(NC * NPAD, 1)
    xs = _tc1(x, deg)
    acc1 = _msg_call(xs, src_p, dst_p)
    h2s = _tc2(acc1, xs, deg, W1, b1.reshape(1, HID2), gamma.reshape(1, HID2),
               beta.reshape(1, HID2), a.reshape(1, 1), W2)
    acc2 = _msg_call(h2s, src_p, dst_p)
    return _tc3(acc2, h2s, deg, b2.reshape(1, OUT_DIM))


# speedup vs baseline: 37.3709x; 1.0088x over previous
import functools

import jax
import jax.numpy as jnp
from jax import lax
from jax.experimental import pallas as pl
from jax.experimental.pallas import tpu as pltpu
from jax.experimental.pallas import tpu_sc as plsc

N = 10000
E = 320000
IN_DIM = 128
HID2 = 256
OUT_DIM = 128
F2 = 128

NC = 2
NS = 16
L = 16

NPAD = 10240
ROWS = NPAD // NS
CH = 128
CHUNKS = 160
TCH = CHUNKS // NC
IB = 16
IBM = 40
EPAD = NS * CHUNKS * CH


def _sc_mesh():
    return plsc.VectorSubcoreMesh(core_axis_name="c", subcore_axis_name="s")


@functools.partial(
    pl.kernel,
    out_type=jax.ShapeDtypeStruct((NC * NPAD,), jnp.float32),
    mesh=_sc_mesh(),
    scratch_types=[
        pltpu.VMEM((IB, CH), jnp.int32),
        pltpu.VMEM((CH,), jnp.float32),
        pltpu.VMEM((ROWS,), jnp.float32),
        pltpu.VMEM_SHARED((NPAD,), jnp.float32),
        pltpu.SemaphoreType.DMA,
        pltpu.SemaphoreType.DMA,
    ],
)
def _deg_call(dst_hbm, out_hbm, dst_v, ones_v, zb, deg_sh, sem0, sem1):
    c = lax.axis_index("c")
    s = lax.axis_index("s")
    base = s * ROWS
    for i in range(CH // L):
        ones_v[pl.ds(i * L, L)] = jnp.ones((L,), jnp.float32)
    for i in range(ROWS // L):
        zb[pl.ds(i * L, L)] = jnp.zeros((L,), jnp.float32)
    pltpu.sync_copy(zb, deg_sh.at[pl.ds(base, ROWS)])
    plsc.subcore_barrier()
    sems = (sem0, sem1)

    row0 = (c * NS + s) * TCH

    def _block(b, carry):
        pltpu.sync_copy(dst_hbm.at[pl.ds(row0 + b * IB, IB), :], dst_v)
        descs = [None, None]
        for i in range(IB):
            p = i % 2
            if descs[p] is not None:
                descs[p].wait()
            descs[p] = pltpu.async_copy(ones_v, deg_sh.at[dst_v.at[i]],
                                        sems[p], add=True)
        descs[0].wait()
        descs[1].wait()
        return carry

    lax.fori_loop(0, TCH // IB, _block, 0)
    plsc.subcore_barrier()
    pltpu.sync_copy(deg_sh.at[pl.ds(base, ROWS)],
                    out_hbm.at[pl.ds(c * NPAD + base, ROWS)])


@functools.partial(
    pl.kernel,
    out_type=jax.ShapeDtypeStruct((NC * NPAD, F2), jnp.float32),
    mesh=_sc_mesh(),
    scratch_types=[
        pltpu.VMEM((IBM, CH), jnp.int32),
        pltpu.VMEM((IBM, CH), jnp.int32),
        pltpu.VMEM((CH, F2), jnp.float32),
        pltpu.VMEM((CH, F2), jnp.float32),
        pltpu.VMEM_SHARED((NPAD, F2), jnp.float32),
        pltpu.SemaphoreType.DMA,
        pltpu.SemaphoreType.DMA,
        pltpu.SemaphoreType.DMA,
        pltpu.SemaphoreType.DMA,
    ],
)
def _msg_call(h_hbm, src_hbm, dst_hbm, out_hbm, src_v, dst_v, buf_a, buf_b,
              acc_sh, sg0, sg1, ss0, ss1):
    c = lax.axis_index("c")
    s = lax.axis_index("s")
    base = s * ROWS
    row0 = (c * NS + s) * TCH
    bufs = (buf_a, buf_b)
    sgs = (sg0, sg1)
    sss = (ss0, ss1)

    def _zrow(i, carry):
        for f in range(F2 // L):
            buf_a[i, pl.ds(f * L, L)] = jnp.zeros((L,), jnp.float32)
        return carry

    lax.fori_loop(0, CH, _zrow, 0)
    for r in range(ROWS // CH):
        pltpu.sync_copy(buf_a, acc_sh.at[pl.ds(base + r * CH, CH), :])
    plsc.subcore_barrier()

    def _block(b, carry):
        pltpu.sync_copy(src_hbm.at[pl.ds(row0 + b * IBM, IBM), :], src_v)
        pltpu.sync_copy(dst_hbm.at[pl.ds(row0 + b * IBM, IBM), :], dst_v)
        gd = [None, None]
        sd = [None, None]
        gd[0] = pltpu.async_copy(h_hbm.at[src_v.at[0]], bufs[0], sgs[0])
        for i in range(IBM):
            p, q = i % 2, (i + 1) % 2
            if i + 1 < IBM:
                if sd[q] is not None:
                    sd[q].wait()
                gd[q] = pltpu.async_copy(h_hbm.at[src_v.at[i + 1]],
                                         bufs[q], sgs[q])
            gd[p].wait()
            sd[p] = pltpu.async_copy(bufs[p], acc_sh.at[dst_v.at[i]],
                                     sss[p], add=True)
        sd[0].wait()
        sd[1].wait()
        return carry

    lax.fori_loop(0, TCH // IBM, _block, 0)
    plsc.subcore_barrier()
    obase = c * NPAD + base
    pltpu.sync_copy(acc_sh.at[pl.ds(base, ROWS), :],
                    out_hbm.at[pl.ds(obase, ROWS), :])


def _dinv(deg_ref):
    return lax.rsqrt(deg_ref[0:N, :] + deg_ref[NPAD:NPAD + N, :] + 1.0)


def _tc1_body(x_ref, deg_ref, out_ref):
    dinv = _dinv(deg_ref)
    out_ref[...] = x_ref[...] * dinv


def _tc2_body(acc_ref, xs_ref, deg_ref, w1_ref, b1_ref, g_ref, be_ref, a_ref,
              w2_ref, out_ref):
    dinv = _dinv(deg_ref)
    agg = (acc_ref[0:N, :] + acc_ref[NPAD:NPAD + N, :] + xs_ref[...]) * dinv
    z = jnp.dot(agg, w1_ref[...], preferred_element_type=jnp.float32)
    z = z + b1_ref[...]
    mean = jnp.sum(z, axis=0, keepdims=True) * (1.0 / N)
    d = z - mean
    var = jnp.sum(d * d, axis=0, keepdims=True) * (1.0 / N)
    zn = d * lax.rsqrt(var + 1e-5) * g_ref[...] + be_ref[...]
    zp = jnp.where(zn > 0, zn, a_ref[0, 0] * zn)
    h2 = jnp.dot(zp, w2_ref[...], preferred_element_type=jnp.float32)
    out_ref[...] = h2 * dinv


def _tc3_body(acc_ref, h_ref, deg_ref, b2_ref, out_ref):
    dinv = _dinv(deg_ref)
    total = acc_ref[0:N, :] + acc_ref[NPAD:NPAD + N, :] + h_ref[...]
    out_ref[...] = total * dinv + b2_ref[...]


_tc1 = pl.pallas_call(
    _tc1_body,
    out_shape=jax.ShapeDtypeStruct((N, IN_DIM), jnp.float32),
)
_tc2 = pl.pallas_call(
    _tc2_body,
    out_shape=jax.ShapeDtypeStruct((N, OUT_DIM), jnp.float32),
)
_tc3 = pl.pallas_call(
    _tc3_body,
    out_shape=jax.ShapeDtypeStruct((N, OUT_DIM), jnp.float32),
)


def kernel(x, edge_index, W1, b1, gamma, beta, a, W2, b2):
    src = edge_index[0].astype(jnp.int32)
    dst = edge_index[1].astype(jnp.int32)
    P = EPAD - E
    pad = jnp.arange(P, dtype=jnp.int32)
    src_p = jnp.concatenate([src, pad % N]).reshape(NS * CHUNKS, CH)
    dst_p = jnp.concatenate([dst, N + pad % (NPAD - N)]).reshape(NS * CHUNKS, CH)

    deg = _deg_call(dst_p).reshape(NC * NPAD, 1)
    xs = _tc1(x, deg)
    acc1 = _msg_call(xs, src_p, dst_p)
    h2s = _tc2(acc1, xs, deg, W1, b1.reshape(1, HID2), gamma.reshape(1, HID2),
               beta.reshape(1, HID2), a.reshape(1, 1), W2)
    acc2 = _msg_call(h2s, src_p, dst_p)
    return _tc3(acc2, h2s, deg, b2.reshape(1, OUT_DIM))


# speedup vs baseline: 37.5755x; 1.0055x over previous
import functools

import jax
import jax.numpy as jnp
from jax import lax
from jax.experimental import pallas as pl
from jax.experimental.pallas import tpu as pltpu
from jax.experimental.pallas import tpu_sc as plsc

N = 10000
E = 320000
IN_DIM = 128
HID2 = 256
OUT_DIM = 128
F2 = 128

NC = 2
NS = 16
L = 16

NPAD = 10240
ROWS = NPAD // NS
CH = 128
NCH = E // CH
TPC = 80
IB = TPC // 2
LASTW = NCH // TPC
MAIN16 = 16
TAIL = 4


def _sc_mesh():
    return plsc.VectorSubcoreMesh(core_axis_name="c", subcore_axis_name="s")


@functools.partial(
    pl.kernel,
    out_type=jax.ShapeDtypeStruct((NC * NPAD,), jnp.float32),
    mesh=_sc_mesh(),
    scratch_types=[
        pltpu.VMEM((IB, CH), jnp.int32),
        pltpu.VMEM((CH,), jnp.float32),
        pltpu.VMEM((ROWS,), jnp.float32),
        pltpu.VMEM_SHARED((NPAD,), jnp.float32),
        pltpu.SemaphoreType.DMA,
        pltpu.SemaphoreType.DMA,
    ],
)
def _deg_call(dst_hbm, dtl_hbm, out_hbm, dst_v, ones_v, zb, deg_sh, sem0,
              sem1):
    c = lax.axis_index("c")
    s = lax.axis_index("s")
    base = s * ROWS
    for i in range(CH // L):
        ones_v[pl.ds(i * L, L)] = jnp.ones((L,), jnp.float32)
    for i in range(ROWS // L):
        zb[pl.ds(i * L, L)] = jnp.zeros((L,), jnp.float32)
    pltpu.sync_copy(zb, deg_sh.at[pl.ds(base, ROWS)])
    plsc.subcore_barrier()
    sems = (sem0, sem1)

    w = c * NS + s
    row0 = w * TPC

    def _count_block(dslice, nrows):
        pltpu.sync_copy(dslice, dst_v.at[pl.ds(0, nrows), :])
        descs = [None, None]
        for i in range(nrows):
            p = i % 2
            if descs[p] is not None:
                descs[p].wait()
            descs[p] = pltpu.async_copy(ones_v, deg_sh.at[dst_v.at[i]],
                                        sems[p], add=True)
        descs[0].wait()
        if descs[1] is not None:
            descs[1].wait()

    @pl.when(w < LASTW)
    def _deg_main():
        for b in range(2):
            _count_block(dst_hbm.at[pl.ds(row0 + b * IB, IB), :], IB)

    @pl.when(w == LASTW)
    def _deg_tail():
        _count_block(dst_hbm.at[pl.ds(row0, MAIN16), :], MAIN16)
        _count_block(dtl_hbm, TAIL)

    plsc.subcore_barrier()
    pltpu.sync_copy(deg_sh.at[pl.ds(base, ROWS)],
                    out_hbm.at[pl.ds(c * NPAD + base, ROWS)])


@functools.partial(
    pl.kernel,
    out_type=jax.ShapeDtypeStruct((NC * NPAD, F2), jnp.float32),
    mesh=_sc_mesh(),
    scratch_types=[
        pltpu.VMEM((IB, CH), jnp.int32),
        pltpu.VMEM((IB, CH), jnp.int32),
        pltpu.VMEM((CH, F2), jnp.float32),
        pltpu.VMEM((CH, F2), jnp.float32),
        pltpu.VMEM_SHARED((NPAD, F2), jnp.float32),
        pltpu.SemaphoreType.DMA,
        pltpu.SemaphoreType.DMA,
        pltpu.SemaphoreType.DMA,
        pltpu.SemaphoreType.DMA,
    ],
)
def _msg_call(h_hbm, src_hbm, dst_hbm, stl_hbm, dtl_hbm, out_hbm, src_v,
              dst_v, buf_a, buf_b, acc_sh, sg0, sg1, ss0, ss1):
    c = lax.axis_index("c")
    s = lax.axis_index("s")
    base = s * ROWS
    w = c * NS + s
    row0 = w * TPC
    bufs = (buf_a, buf_b)
    sgs = (sg0, sg1)
    sss = (ss0, ss1)

    def _zrow(i, carry):
        for f in range(F2 // L):
            buf_a[i, pl.ds(f * L, L)] = jnp.zeros((L,), jnp.float32)
        return carry

    lax.fori_loop(0, CH, _zrow, 0)
    for r in range(ROWS // CH):
        pltpu.sync_copy(buf_a, acc_sh.at[pl.ds(base + r * CH, CH), :])
    plsc.subcore_barrier()

    def _edge_block(sslice, dslice, nrows):
        pltpu.sync_copy(sslice, src_v.at[pl.ds(0, nrows), :])
        pltpu.sync_copy(dslice, dst_v.at[pl.ds(0, nrows), :])
        gd = [None, None]
        sd = [None, None]
        gd[0] = pltpu.async_copy(h_hbm.at[src_v.at[0]], bufs[0], sgs[0])
        for i in range(nrows):
            p, q = i % 2, (i + 1) % 2
            if i + 1 < nrows:
                if sd[q] is not None:
                    sd[q].wait()
                gd[q] = pltpu.async_copy(h_hbm.at[src_v.at[i + 1]],
                                         bufs[q], sgs[q])
            gd[p].wait()
            sd[p] = pltpu.async_copy(bufs[p], acc_sh.at[dst_v.at[i]],
                                     sss[p], add=True)
        sd[0].wait()
        if sd[1] is not None:
            sd[1].wait()

    @pl.when(w < LASTW)
    def _msg_main():
        for b in range(2):
            _edge_block(src_hbm.at[pl.ds(row0 + b * IB, IB), :],
                        dst_hbm.at[pl.ds(row0 + b * IB, IB), :], IB)

    @pl.when(w == LASTW)
    def _msg_tail():
        _edge_block(src_hbm.at[pl.ds(row0, MAIN16), :],
                    dst_hbm.at[pl.ds(row0, MAIN16), :], MAIN16)
        _edge_block(stl_hbm, dtl_hbm, TAIL)

    plsc.subcore_barrier()
    obase = c * NPAD + base
    pltpu.sync_copy(acc_sh.at[pl.ds(base, ROWS), :],
                    out_hbm.at[pl.ds(obase, ROWS), :])


def _dinv(deg_ref):
    return lax.rsqrt(deg_ref[0:N, :] + deg_ref[NPAD:NPAD + N, :] + 1.0)


def _tc1_body(x_ref, deg_ref, out_ref):
    dinv = _dinv(deg_ref)
    out_ref[...] = x_ref[...] * dinv


def _tc2_body(acc_ref, xs_ref, deg_ref, w1_ref, b1_ref, g_ref, be_ref, a_ref,
              w2_ref, out_ref):
    dinv = _dinv(deg_ref)
    agg = (acc_ref[0:N, :] + acc_ref[NPAD:NPAD + N, :] + xs_ref[...]) * dinv
    z = jnp.dot(agg, w1_ref[...], preferred_element_type=jnp.float32)
    z = z + b1_ref[...]
    mean = jnp.sum(z, axis=0, keepdims=True) * (1.0 / N)
    d = z - mean
    var = jnp.sum(d * d, axis=0, keepdims=True) * (1.0 / N)
    zn = d * lax.rsqrt(var + 1e-5) * g_ref[...] + be_ref[...]
    zp = jnp.where(zn > 0, zn, a_ref[0, 0] * zn)
    h2 = jnp.dot(zp, w2_ref[...], preferred_element_type=jnp.float32)
    out_ref[...] = h2 * dinv


def _tc3_body(acc_ref, h_ref, deg_ref, b2_ref, out_ref):
    dinv = _dinv(deg_ref)
    total = acc_ref[0:N, :] + acc_ref[NPAD:NPAD + N, :] + h_ref[...]
    out_ref[...] = total * dinv + b2_ref[...]


_tc1 = pl.pallas_call(
    _tc1_body,
    out_shape=jax.ShapeDtypeStruct((N, IN_DIM), jnp.float32),
)
_tc2 = pl.pallas_call(
    _tc2_body,
    out_shape=jax.ShapeDtypeStruct((N, OUT_DIM), jnp.float32),
)
_tc3 = pl.pallas_call(
    _tc3_body,
    out_shape=jax.ShapeDtypeStruct((N, OUT_DIM), jnp.float32),
)


def kernel(x, edge_index, W1, b1, gamma, beta, a, W2, b2):
    src_f = edge_index[0].astype(jnp.int32)
    dst_f = edge_index[1].astype(jnp.int32)
    src_p = src_f.reshape(NCH, CH)
    dst_p = dst_f.reshape(NCH, CH)
    src_t = src_f[E - TAIL * CH:].reshape(TAIL, CH)
    dst_t = dst_f[E - TAIL * CH:].reshape(TAIL, CH)

    deg = _deg_call(dst_p, dst_t).reshape(NC * NPAD, 1)
    xs = _tc1(x, deg)
    acc1 = _msg_call(xs, src_p, dst_p, src_t, dst_t)
    h2s = _tc2(acc1, xs, deg, W1, b1.reshape(1, HID2), gamma.reshape(1, HID2),
               beta.reshape(1, HID2), a.reshape(1, 1), W2)
    acc2 = _msg_call(h2s, src_p, dst_p, src_t, dst_t)
    return _tc3(acc2, h2s, deg, b2.reshape(1, OUT_DIM))
